# baseline (speedup 1.0000x reference)
import functools

import jax
import jax.numpy as jnp
from jax import lax
from jax.experimental import pallas as pl
from jax.experimental.pallas import tpu as pltpu
from jax.experimental.pallas import tpu_sc as plsc

B, F = 16384, 100
NC, NS, L = 2, 16, 16
NW = NC * NS


@functools.partial(
    pl.kernel,
    mesh=plsc.VectorSubcoreMesh(core_axis_name="c", subcore_axis_name="s"),
    out_type=jax.ShapeDtypeStruct((F, 2 * (B // 128), 128), jnp.float32),
    scratch_types=[
        pltpu.VMEM((4, 8, 128), jnp.int32),
        pltpu.VMEM((8, 8, 128), jnp.float32),
    ],
    compiler_params=pltpu.CompilerParams(needs_layout_passes=False),
)
def _onehot_sc2(x_hbm, out_hbm, xin, oout):
    wid = lax.axis_index("s") * NC + lax.axis_index("c")
    b0 = wid * 512

    for i in range(13):
        f0 = i * 8
        nf = min(8, F - f0)
        for j in range(4):
            pltpu.sync_copy(
                x_hbm.at[pl.ds(f0, nf), pl.ds(b0 + 128 * j, 128)],
                xin.at[j, pl.ds(0, nf)],
            )

        def f_body(s, carry):
            for j in range(4):
                for t in range(128 // L):
                    x = xin[j, s, pl.ds(t * L, L)]
                    v = x.astype(jnp.float32)
                    oout[s, 2 * j, pl.ds(t * L, L)] = 1.0 - v
                    oout[s, 2 * j + 1, pl.ds(t * L, L)] = v
            return carry

        lax.fori_loop(0, nf, f_body, 0)

        pltpu.sync_copy(
            oout.at[pl.ds(0, nf)],
            out_hbm.at[pl.ds(f0, nf), pl.ds(8 * wid, 8), :],
        )


def kernel(inputs):
    xt = inputs.astype(jnp.int32).T
    o3 = _onehot_sc2(xt)
    o4 = o3.reshape(F, B // 128, 2, 128)
    return o4.transpose(1, 3, 0, 2).reshape(B, F, 2)


# speedup vs baseline: 1.7956x; 1.7956x over previous
import functools

import jax
import jax.numpy as jnp
from jax import lax
from jax.experimental import pallas as pl
from jax.experimental.pallas import tpu as pltpu
from jax.experimental.pallas import tpu_sc as plsc

B, F = 16384, 100
NC, NS, L = 2, 16, 16
NW = NC * NS
NT = 13


@functools.partial(
    pl.kernel,
    mesh=plsc.VectorSubcoreMesh(core_axis_name="c", subcore_axis_name="s"),
    out_type=jax.ShapeDtypeStruct((F, 2 * (B // 128), 128), jnp.float32),
    scratch_types=[
        pltpu.VMEM((2, 8, 512), jnp.int32),
        pltpu.VMEM((2, 8, 8, 128), jnp.float32),
        pltpu.SemaphoreType.DMA,
        pltpu.SemaphoreType.DMA,
    ],
)
def _onehot_sc2(x_hbm, out_hbm, xin, oout, in_sem, out_sem):
    wid = lax.axis_index("s") * NC + lax.axis_index("c")
    b0 = wid * 512

    def nf(i):
        return min(8, F - i * 8)

    def start_in(i):
        return pltpu.async_copy(
            x_hbm.at[pl.ds(i * 8, nf(i)), pl.ds(b0, 512)],
            xin.at[i % 2, pl.ds(0, nf(i))],
            in_sem,
        )

    def start_out(i):
        return pltpu.async_copy(
            oout.at[i % 2, pl.ds(0, nf(i))],
            out_hbm.at[pl.ds(i * 8, nf(i)), pl.ds(8 * wid, 8), :],
            out_sem,
        )

    in_copies = {0: start_in(0)}
    out_copies = {}
    for i in range(NT):
        if i + 1 < NT:
            in_copies[i + 1] = start_in(i + 1)
        in_copies.pop(i).wait()
        if i >= 2:
            out_copies.pop(i - 2).wait()

        def f_body(s, carry):
            for j in range(4):
                for t in range(128 // L):
                    x = xin[i % 2, s, pl.ds(j * 128 + t * L, L)]
                    v = x.astype(jnp.float32)
                    oout[i % 2, s, 2 * j, pl.ds(t * L, L)] = 1.0 - v
                    oout[i % 2, s, 2 * j + 1, pl.ds(t * L, L)] = v
            return carry

        lax.fori_loop(0, nf(i), f_body, 0)
        out_copies[i] = start_out(i)

    for i in sorted(out_copies):
        out_copies.pop(i).wait()


def kernel(inputs):
    xt = inputs.astype(jnp.int32).T
    o3 = _onehot_sc2(xt)
    o4 = o3.reshape(F, B // 128, 2, 128)
    return o4.transpose(1, 3, 0, 2).reshape(B, F, 2)


# speedup vs baseline: 9.7901x; 5.4522x over previous
import functools

import jax
import jax.numpy as jnp
from jax.experimental import pallas as pl
from jax.experimental.pallas import tpu as pltpu

B, F = 16384, 100
FB = 48
BB = 16384
GRID_F = (F + FB - 1) // FB
GRID_B = B // BB


def _block(x_ref, o_ref):
    v = x_ref[...].astype(jnp.float32)
    v3 = v.reshape(FB, BB // 128, 128)
    u3 = 1.0 - v3
    o_ref[:, ::2, :] = u3
    o_ref[:, 1::2, :] = v3


_onehot = pl.pallas_call(
    _block,
    grid=(GRID_F, GRID_B),
    in_specs=[pl.BlockSpec((FB, BB), lambda i, j: (i, j))],
    out_specs=pl.BlockSpec((FB, 2 * (BB // 128), 128), lambda i, j: (i, j, 0)),
    out_shape=jax.ShapeDtypeStruct((F, 2 * (B // 128), 128), jnp.float32),
)


def kernel(inputs):
    xt = inputs.astype(jnp.int32).T
    o3 = _onehot(xt)
    o4 = o3.reshape(F, B // 128, 2, 128)
    return o4.transpose(1, 3, 0, 2).reshape(B, F, 2)


# speedup vs baseline: 9.9176x; 1.0130x over previous
import functools

import jax
import jax.numpy as jnp
from jax.experimental import pallas as pl
from jax.experimental.pallas import tpu as pltpu

B, F = 16384, 100
FB = 40
BB = 16384
GRID_F = (F + FB - 1) // FB
GRID_B = B // BB


def _block(x_ref, o_ref):
    v = x_ref[...].astype(jnp.float32)
    v3 = v.reshape(FB, BB // 128, 128)
    u3 = 1.0 - v3
    o_ref[:, ::2, :] = u3
    o_ref[:, 1::2, :] = v3


_onehot = pl.pallas_call(
    _block,
    grid=(GRID_F, GRID_B),
    in_specs=[pl.BlockSpec((FB, BB), lambda i, j: (i, j))],
    out_specs=pl.BlockSpec((FB, 2 * (BB // 128), 128), lambda i, j: (i, j, 0)),
    out_shape=jax.ShapeDtypeStruct((F, 2 * (B // 128), 128), jnp.float32),
)


def kernel(inputs):
    xt = inputs.astype(jnp.int32).T
    o3 = _onehot(xt)
    o4 = o3.reshape(F, B // 128, 2, 128)
    return o4.transpose(1, 3, 0, 2).reshape(B, F, 2)


# speedup vs baseline: 9.9522x; 1.0035x over previous
import jax
import jax.numpy as jnp
from jax.experimental import pallas as pl

B, F = 16384, 100
FB = 40
BB = 16384
GRID_F = (F + FB - 1) // FB
GRID_B = B // BB


def _block(x_ref, o_ref):
    v = x_ref[...].astype(jnp.float32)
    v3 = v.reshape(FB, BB // 128, 128)
    u3 = 1.0 - v3
    o_ref[:, ::2, :] = u3
    o_ref[:, 1::2, :] = v3


_onehot = pl.pallas_call(
    _block,
    grid=(GRID_F, GRID_B),
    in_specs=[pl.BlockSpec((FB, BB), lambda i, j: (i, j))],
    out_specs=pl.BlockSpec((FB, 2 * (BB // 128), 128), lambda i, j: (i, j, 0)),
    out_shape=jax.ShapeDtypeStruct((F, 2 * (B // 128), 128), jnp.float32),
)


def kernel(inputs):
    xt = inputs.astype(jnp.int32).T
    o3 = _onehot(xt)
    o4 = o3.reshape(F, B // 128, 2, 128)
    return o4.transpose(1, 3, 0, 2).reshape(B, F, 2)
